# baseline (speedup 1.0000x reference)
import functools

import jax
import jax.numpy as jnp
from jax import lax
from jax.experimental import pallas as pl
from jax.experimental.pallas import tpu as pltpu
from jax.experimental.pallas import tpu_sc as plsc

N = 25000
E = 800000
NCORE, NSUB = 2, 16
NW = NCORE * NSUB

NPAD = 25088
RPT = NPAD // NSUB
ECHUNK = 128
CPT = 392
NCHUNK = CPT * NSUB + 2
EPAD = NCHUNK * ECHUNK
CW = 8

IPAD = 28672
RB = 1000

@functools.lru_cache(maxsize=None)
def _sc_mesh():
    return plsc.VectorSubcoreMesh(
        core_axis_name="c", subcore_axis_name="s",
        num_cores=NCORE, num_subcores=NSUB)


def _emb_body(idx2d, tab, out, idxv, rows, sem):
    wid = lax.axis_index("s") * NCORE + lax.axis_index("c")
    pltpu.sync_copy(idx2d.at[pl.ds(wid * 16, 16)], idxv)
    for j in range(16):
        pltpu.async_copy(
            tab.at[idxv.at[j]], rows.at[pl.ds(j * 112, 112)], sem)
    for j in range(16):
        pltpu.make_async_copy(
            tab.at[idxv.at[j]], rows.at[pl.ds(j * 112, 112)], sem).wait()
    pltpu.sync_copy(rows, out.at[pl.ds(wid * 1792, 1792)])


@functools.lru_cache(maxsize=None)
def _emb_kernel():
    return pl.kernel(
        _emb_body,
        out_type=jax.ShapeDtypeStruct((2 * IPAD, 16), jnp.float32),
        mesh=_sc_mesh(),
        compiler_params=pltpu.CompilerParams(use_tc_tiling_on_sc=False),
        scratch_types=[
            pltpu.VMEM((16, 112), jnp.int32),
            pltpu.VMEM((1792, 16), jnp.float32),
            pltpu.SemaphoreType.DMA,
        ],
    )


def _emb_call(idx2d, tab):
    return _emb_kernel()(idx2d, tab)


def _seg_body(tab_c, tab_p, srcC, dstC, srcP, dstP, zsum, zcnt, onesr,
              osum_c, ocnt_c, osum_p, ocnt_p,
              acc_sum, acc_cnt, sidx, didx, rows, onesv,
              gsem0, gsem1, isem, idsem, ssem0, ssem1):
    cid = lax.axis_index("c")
    sid = lax.axis_index("s")
    r0 = sid * RPT

    pltpu.sync_copy(zsum, acc_sum.at[pl.ds(r0, RPT)])
    pltpu.sync_copy(zcnt, acc_cnt.at[pl.ds(r0, RPT)])
    pltpu.sync_copy(onesr, onesv)
    plsc.subcore_barrier()

    def run(src2d, dst2d, tab, osum, ocnt):
        base = sid * CPT
        ssems = (ssem0, ssem1)
        gsems = (gsem0, gsem1)

        def step(c_ref, j, p, first):
            q = 1 - p
            def waits():
                pltpu.make_async_copy(
                    rows.at[q], acc_sum.at[didx.at[q]], ssems[q]).wait()
                pltpu.make_async_copy(
                    onesv, acc_cnt.at[didx.at[q]], ssems[q]).wait()
                pltpu.make_async_copy(
                    dst2d.at[c_ref], didx.at[p], idsem).wait()

            if first:
                @pl.when(j > 0)
                def _():
                    waits()
            else:
                waits()
            pltpu.async_copy(dst2d.at[c_ref + 1], didx.at[q], idsem)
            pltpu.make_async_copy(src2d.at[c_ref + 1], sidx.at[q],
                                  isem).wait()
            pltpu.async_copy(tab.at[sidx.at[q]], rows.at[q], gsems[q])
            pltpu.make_async_copy(
                tab.at[sidx.at[p]], rows.at[p], gsems[p]).wait()
            pltpu.async_copy(src2d.at[c_ref + 2], sidx.at[p], isem)
            pltpu.async_copy(rows.at[p], acc_sum.at[didx.at[p]], ssems[p],
                             add=True)
            pltpu.async_copy(onesv, acc_cnt.at[didx.at[p]], ssems[p],
                             add=True)

        pltpu.sync_copy(src2d.at[base], sidx.at[0])
        pltpu.sync_copy(dst2d.at[base], didx.at[0])
        pltpu.async_copy(tab.at[sidx.at[0]], rows.at[0], gsem0)
        pltpu.async_copy(src2d.at[base + 1], sidx.at[1], isem)

        def pair(j, carry):
            c0 = base + 2 * j
            step(c0, j, 0, True)
            step(c0 + 1, j, 1, False)
            return carry

        lax.fori_loop(0, CPT // 2, pair, 0)

        pltpu.make_async_copy(tab.at[sidx.at[0]], rows.at[0], gsem0).wait()
        pltpu.make_async_copy(rows.at[1], acc_sum.at[didx.at[1]], ssem1).wait()
        pltpu.make_async_copy(onesv, acc_cnt.at[didx.at[1]], ssem1).wait()
        pltpu.make_async_copy(src2d.at[base], sidx.at[1], isem).wait()
        pltpu.make_async_copy(dst2d.at[base], didx.at[1], idsem).wait()

        plsc.subcore_barrier()
        pltpu.sync_copy(acc_sum.at[pl.ds(r0, RPT)], osum.at[pl.ds(r0, RPT)])
        pltpu.sync_copy(acc_cnt.at[pl.ds(r0, RPT)], ocnt.at[pl.ds(r0, RPT)])

    @pl.when(cid == 0)
    def _():
        run(srcC, dstC, tab_p, osum_c, ocnt_c)

    @pl.when(cid == 1)
    def _():
        run(srcP, dstP, tab_c, osum_p, ocnt_p)


@functools.lru_cache(maxsize=None)
def _seg_kernel():
    return pl.kernel(
        _seg_body,
        out_type=(jax.ShapeDtypeStruct((NPAD, 64), jnp.float32),
                  jax.ShapeDtypeStruct((NPAD, CW), jnp.float32),
                  jax.ShapeDtypeStruct((NPAD, 64), jnp.float32),
                  jax.ShapeDtypeStruct((NPAD, CW), jnp.float32)),
        mesh=_sc_mesh(),
        compiler_params=pltpu.CompilerParams(use_tc_tiling_on_sc=False),
        scratch_types=[
            pltpu.VMEM_SHARED((NPAD, 64), jnp.float32),
            pltpu.VMEM_SHARED((NPAD, CW), jnp.float32),
            pltpu.VMEM((2, ECHUNK), jnp.int32),
            pltpu.VMEM((2, ECHUNK), jnp.int32),
            pltpu.VMEM((2, ECHUNK, 64), jnp.float32),
            pltpu.VMEM((ECHUNK, CW), jnp.float32),
            pltpu.SemaphoreType.DMA,
            pltpu.SemaphoreType.DMA,
            pltpu.SemaphoreType.DMA,
            pltpu.SemaphoreType.DMA,
            pltpu.SemaphoreType.DMA,
            pltpu.SemaphoreType.DMA,
        ],
    )


def _seg_call(tab_c, tab_p, srcC2, dstC2, srcP2, dstP2, zsum, zcnt, onesr):
    return _seg_kernel()(
        tab_c, tab_p, srcC2, dstC2, srcP2, dstP2, zsum, zcnt, onesr)


def _dot(a, b):
    return jnp.dot(a, b, preferred_element_type=jnp.float32)


def _accum_stats(b, acc, st, hs):
    @pl.when(b == 0)
    def _():
        acc[...] = jnp.zeros((4, 64), jnp.float32)

    hc, hp = hs
    acc[0:1, :] += jnp.sum(hc, axis=0, keepdims=True)
    acc[1:2, :] += jnp.sum(hc * hc, axis=0, keepdims=True)
    acc[2:3, :] += jnp.sum(hp, axis=0, keepdims=True)
    acc[3:4, :] += jnp.sum(hp * hp, axis=0, keepdims=True)
    st[...] = acc[...]


def _a1_body(xc, ec, xp, ep, wca, wce, bca, wpa, wpe, bpa,
             h0c, c0c, h0p, c0p, st, acc):
    def node_type(x, e, wa, we, b, ho, co):
        pre = _dot(x[...], wa[...]) + _dot(e[...], we[...]) + b[...]
        c0 = jax.nn.sigmoid(pre[:, 0:64]) * jnp.tanh(pre[:, 64:128])
        h0 = jax.nn.sigmoid(pre[:, 128:192]) * jnp.tanh(c0)
        co[...] = c0
        ho[...] = h0
        return h0

    h_c = node_type(xc, ec, wca, wce, bca, h0c, c0c)
    h_p = node_type(xp, ep, wpa, wpe, bpa, h0p, c0p)
    _accum_stats(pl.program_id(0), acc, st, (h_c, h_p))


def _graph_norm(h, prm, st, k):
    ms = prm[3 * k:3 * k + 1, :]
    w = prm[3 * k + 1:3 * k + 2, :]
    b = prm[3 * k + 2:3 * k + 3, :]
    inv_n = jnp.float32(1.0 / N)
    m = st[2 * k:2 * k + 1, :] * inv_n
    msq = st[2 * k + 1:2 * k + 2, :] * inv_n
    var = msq - m * m * (2.0 * ms - ms * ms)
    out = h - ms * m
    return out / jnp.sqrt(var + 1e-5) * w + b


def _b1_body(h0c, c0c, sc, cc, h0p, c0p, sp, cp_, wxc, wmc, bc, wxp, wmp, bp,
             nrm0, st0, c1c, h1c, c1p, h1p, st, acc):
    def affine(k):
        ms = nrm0[3 * k:3 * k + 1, :]
        w = nrm0[3 * k + 1:3 * k + 2, :]
        bb = nrm0[3 * k + 2:3 * k + 3, :]
        inv_n = jnp.float32(1.0 / N)
        m = st0[2 * k:2 * k + 1, :] * inv_n
        msq = st0[2 * k + 1:2 * k + 2, :] * inv_n
        var = msq - m * m * (2.0 * ms - ms * ms)
        istd = 1.0 / jnp.sqrt(var + 1e-5)
        alpha = istd * w
        return alpha, bb - ms * m * alpha

    def node_type(h0, c0, s, cnt, wx, wm, b, c1o, h1o, k):
        alpha, beta = affine(k)
        alpha_s, beta_s = affine(1 - k)
        x1 = h0[...] * alpha + beta
        cntv = cnt[...][:, 0:1]
        inv = 1.0 / jnp.maximum(cntv, 1.0)
        nonempty = cntv * inv
        mean = s[...] * inv * alpha_s + nonempty * beta_s
        pre = _dot(x1, wx[...]) + _dot(mean, wm[...]) + b[...]
        c1 = (jax.nn.sigmoid(pre[:, 64:128]) * c0[...]
              + jax.nn.sigmoid(pre[:, 0:64]) * jnp.tanh(pre[:, 128:192]))
        h1 = jax.nn.sigmoid(pre[:, 192:256]) * jnp.tanh(c1)
        c1o[...] = c1
        h1o[...] = h1
        return h1

    h_c = node_type(h0c, c0c, sc, cc, wxc, wmc, bc, c1c, h1c, 0)
    h_p = node_type(h0p, c0p, sp, cp_, wxp, wmp, bp, c1p, h1p, 1)
    _accum_stats(pl.program_id(0), acc, st, (h_c, h_p))


def _b2_body(h1c, h1p, nrm, st, wvt, bv, wot, bo, w1, b1, w2, b2,
             pred, hc, hp):
    xc = _graph_norm(h1c[...], nrm[...], st[...], 0)
    hp[...] = _graph_norm(h1p[...], nrm[...], st[...], 1)
    hc[...] = xc
    v = _dot(xc, wvt[...]) + bv[...]
    att = xc + _dot(v, wot[...]) + bo[...]
    z = jnp.maximum(_dot(att, w1[...]) + b1[...], 0.0)
    pred[...] = _dot(z, w2[...]) + b2[...]


def _full(shape):
    return pl.BlockSpec(shape, lambda b: (0, 0))


def _rows(width):
    return pl.BlockSpec((RB, width), lambda b: (b, 0))


def kernel(x_customer, x_product, edge_index_buys, edge_index_rev_buys, params):
    p = params
    f32 = jnp.float32

    country_ids = x_customer[:, -1].astype(jnp.int32)
    desc_ids = x_product[:, -1].astype(jnp.int32)
    x14 = x_customer[:, :-1]
    x4 = x_product[:, :-1]
    ctab = jnp.pad(p['country_embed'], ((0, 0), (0, 12)))
    tab = jnp.concatenate([ctab, p['desc_embed']])
    idx2d = jnp.concatenate([
        jnp.pad(country_ids, (0, IPAD - N)),
        jnp.pad(desc_ids + ctab.shape[0], (0, IPAD - N)),
    ]).reshape(NW * 16, 112)

    emb = _emb_call(idx2d, tab)
    emb_c = emb[:N]
    emb_p = emb[IPAD:IPAD + N]

    G0 = ('i', 'c', 'o')
    L0 = p['layers'][0]
    wca = jnp.concatenate([L0['W_' + g]['customer'][:14] for g in G0], axis=1)
    wce = jnp.pad(
        jnp.concatenate([L0['W_' + g]['customer'][14:] for g in G0], axis=1),
        ((0, 12), (0, 0)))
    bca = jnp.concatenate(
        [L0['b_' + g]['customer'] + L0['conv_' + g]['rev']['bl'][None, :]
         for g in G0], axis=1)
    wpa = jnp.concatenate([L0['W_' + g]['product'][:4] for g in G0], axis=1)
    wpe = jnp.concatenate([L0['W_' + g]['product'][4:] for g in G0], axis=1)
    bpa = jnp.concatenate(
        [L0['b_' + g]['product'] + L0['conv_' + g]['buys']['bl'][None, :]
         for g in G0], axis=1)

    h0c, c0c, h0p, c0p, st0 = pl.pallas_call(
        _a1_body,
        grid=(N // RB,),
        in_specs=[
            _rows(14), _rows(16), _rows(4), _rows(16),
            _full((14, 192)), _full((16, 192)), _full((1, 192)),
            _full((4, 192)), _full((16, 192)), _full((1, 192)),
        ],
        out_specs=[_rows(64)] * 4 + [_full((4, 64))],
        out_shape=[jax.ShapeDtypeStruct((N, 64), f32)] * 4
        + [jax.ShapeDtypeStruct((4, 64), f32)],
        scratch_shapes=[pltpu.VMEM((4, 64), f32)],
    )(x14, emb_c, x4, emb_p, wca, wce, bca, wpa, wpe, bpa)

    def norm_pack(layer):
        return jnp.stack([
            layer['norm']['customer']['mean_scale'],
            layer['norm']['customer']['weight'],
            layer['norm']['customer']['bias'],
            layer['norm']['product']['mean_scale'],
            layer['norm']['product']['weight'],
            layer['norm']['product']['bias'],
        ])

    def prep_edges(src, dst):
        s = jnp.concatenate([src, jnp.zeros((EPAD - E,), jnp.int32)])
        d = jnp.concatenate([dst, jnp.full((EPAD - E,), N, jnp.int32)])
        return s.reshape(NCHUNK, ECHUNK), d.reshape(NCHUNK, ECHUNK)

    srcC2, dstC2 = prep_edges(edge_index_rev_buys[0], edge_index_rev_buys[1])
    srcP2, dstP2 = prep_edges(edge_index_buys[0], edge_index_buys[1])
    zsum = jnp.zeros((RPT, 64), f32)
    zcnt = jnp.zeros((RPT, CW), f32)
    onesr = jnp.ones((ECHUNK, CW), f32)

    sum_c, cnt_c, sum_p, cnt_p = _seg_call(
        h0c, h0p, srcC2, dstC2, srcP2, dstP2, zsum, zcnt, onesr)

    sum_c, cnt_c = sum_c[:N], cnt_c[:N]
    sum_p, cnt_p = sum_p[:N], cnt_p[:N]

    L1 = p['layers'][1]
    G1 = ('i', 'f', 'c', 'o')

    def wpack(nt, et):
        wx = jnp.concatenate(
            [L1['W_' + g][nt] + L1['conv_' + g][et]['Wr'] for g in G1], axis=1)
        wm = jnp.concatenate([L1['conv_' + g][et]['Wl'] for g in G1], axis=1)
        b = jnp.concatenate(
            [L1['b_' + g][nt] + L1['conv_' + g][et]['bl'][None, :]
             for g in G1], axis=1)
        return wx, wm, b

    wxc, wmc, bc1 = wpack('customer', 'rev')
    wxp, wmp, bp1 = wpack('product', 'buys')

    c1c, h1c, c1p, h1p, st1 = pl.pallas_call(
        _b1_body,
        grid=(N // RB,),
        in_specs=[
            _rows(64), _rows(64), _rows(64), _rows(CW),
            _rows(64), _rows(64), _rows(64), _rows(CW),
            _full((64, 256)), _full((64, 256)), _full((1, 256)),
            _full((64, 256)), _full((64, 256)), _full((1, 256)),
            _full((6, 64)), _full((4, 64)),
        ],
        out_specs=[_rows(64)] * 4 + [_full((4, 64))],
        out_shape=[jax.ShapeDtypeStruct((N, 64), f32)] * 4
        + [jax.ShapeDtypeStruct((4, 64), f32)],
        scratch_shapes=[pltpu.VMEM((4, 64), f32)],
    )(h0c, c0c, sum_c, cnt_c, h0p, c0p, sum_p, cnt_p,
      wxc, wmc, bc1, wxp, wmp, bp1, norm_pack(L0), st0)

    wvt = p['mha']['in_w'][128:192].T
    bv = p['mha']['in_b'][128:192][None, :]
    wot = p['mha']['out_w'].T
    bo = p['mha']['out_b'][None, :]
    w1 = p['pred']['W1']
    b1 = p['pred']['b1'][None, :]
    w2 = jnp.pad(p['pred']['W2'], ((0, 0), (0, 7)))
    b2 = jnp.pad(p['pred']['b2'], (0, 7))[None, :]

    pred8, hc, hp = pl.pallas_call(
        _b2_body,
        grid=(N // RB,),
        in_specs=[
            _rows(64), _rows(64), _full((6, 64)), _full((4, 64)),
            _full((64, 64)), _full((1, 64)), _full((64, 64)), _full((1, 64)),
            _full((64, 32)), _full((1, 32)), _full((32, 8)), _full((1, 8)),
        ],
        out_specs=[_rows(8), _rows(64), _rows(64)],
        out_shape=[jax.ShapeDtypeStruct((N, 8), f32),
                   jax.ShapeDtypeStruct((N, 64), f32),
                   jax.ShapeDtypeStruct((N, 64), f32)],
    )(h1c, h1p, norm_pack(L1), st1, wvt, bv, wot, bo, w1, b1, w2, b2)

    return (pred8[:, :1], hc, hp, c1c, c1p)


# speedup vs baseline: 1.0085x; 1.0085x over previous
import functools

import jax
import jax.numpy as jnp
from jax import lax
from jax.experimental import pallas as pl
from jax.experimental.pallas import tpu as pltpu
from jax.experimental.pallas import tpu_sc as plsc

N = 25000
E = 800000
NCORE, NSUB = 2, 16
NW = NCORE * NSUB

NPAD = 25088
RPT = NPAD // NSUB
ECHUNK = 128
CPT = 392
NCHUNK = CPT * NSUB + 2
EPAD = NCHUNK * ECHUNK
CW = 8

IPAD = 28672
RB = 1000

@functools.lru_cache(maxsize=None)
def _sc_mesh():
    return plsc.VectorSubcoreMesh(
        core_axis_name="c", subcore_axis_name="s",
        num_cores=NCORE, num_subcores=NSUB)


def _emb_body(idx2d, tab, out, idxv, rows, sem):
    wid = lax.axis_index("s") * NCORE + lax.axis_index("c")
    pltpu.sync_copy(idx2d.at[pl.ds(wid * 16, 16)], idxv)
    for j in range(16):
        pltpu.async_copy(
            tab.at[idxv.at[j]], rows.at[pl.ds(j * 112, 112)], sem)
    for j in range(16):
        pltpu.make_async_copy(
            tab.at[idxv.at[j]], rows.at[pl.ds(j * 112, 112)], sem).wait()
    pltpu.sync_copy(rows, out.at[pl.ds(wid * 1792, 1792)])


@functools.lru_cache(maxsize=None)
def _emb_kernel():
    return pl.kernel(
        _emb_body,
        out_type=jax.ShapeDtypeStruct((2 * IPAD, 16), jnp.float32),
        mesh=_sc_mesh(),
        compiler_params=pltpu.CompilerParams(use_tc_tiling_on_sc=False),
        scratch_types=[
            pltpu.VMEM((16, 112), jnp.int32),
            pltpu.VMEM((1792, 16), jnp.float32),
            pltpu.SemaphoreType.DMA,
        ],
    )


def _emb_call(idx2d, tab):
    return _emb_kernel()(idx2d, tab)


def _seg_body(tab_c, tab_p, srcC, dstC, srcP, dstP, zsum, zcnt, onesr,
              osum_c, ocnt_c, osum_p, ocnt_p,
              acc_sum, acc_cnt, sidx, didx, rows, onesv,
              gsem0, gsem1, isem, idsem, ssem0, ssem1):
    cid = lax.axis_index("c")
    sid = lax.axis_index("s")
    r0 = sid * RPT

    pltpu.sync_copy(zsum, acc_sum.at[pl.ds(r0, RPT)])
    pltpu.sync_copy(zcnt, acc_cnt.at[pl.ds(r0, RPT)])
    pltpu.sync_copy(onesr, onesv)
    plsc.subcore_barrier()

    def run(src2d, dst2d, tab, osum, ocnt):
        base = sid * CPT
        ssems = (ssem0, ssem1)
        gsems = (gsem0, gsem1)

        def step(c_ref, j, p, first):
            q = 1 - p
            def waits():
                pltpu.make_async_copy(
                    rows.at[q], acc_sum.at[didx.at[q]], ssems[q]).wait()
                pltpu.make_async_copy(
                    onesv, acc_cnt.at[didx.at[q]], ssems[q]).wait()
                pltpu.make_async_copy(
                    dst2d.at[c_ref], didx.at[p], idsem).wait()

            if first:
                @pl.when(j > 0)
                def _():
                    waits()
            else:
                waits()
            pltpu.async_copy(dst2d.at[c_ref + 1], didx.at[q], idsem)
            pltpu.make_async_copy(src2d.at[c_ref + 1], sidx.at[q],
                                  isem).wait()
            pltpu.async_copy(tab.at[sidx.at[q]], rows.at[q], gsems[q])
            pltpu.make_async_copy(
                tab.at[sidx.at[p]], rows.at[p], gsems[p]).wait()
            pltpu.async_copy(src2d.at[c_ref + 2], sidx.at[p], isem)
            pltpu.async_copy(rows.at[p], acc_sum.at[didx.at[p]], ssems[p],
                             add=True)
            pltpu.async_copy(onesv, acc_cnt.at[didx.at[p]], ssems[p],
                             add=True)

        pltpu.sync_copy(src2d.at[base], sidx.at[0])
        pltpu.sync_copy(dst2d.at[base], didx.at[0])
        pltpu.async_copy(tab.at[sidx.at[0]], rows.at[0], gsem0)
        pltpu.async_copy(src2d.at[base + 1], sidx.at[1], isem)

        def pair(j, carry):
            c0 = base + 2 * j
            step(c0, j, 0, True)
            step(c0 + 1, j, 1, False)
            return carry

        lax.fori_loop(0, CPT // 2, pair, 0)

        pltpu.make_async_copy(tab.at[sidx.at[0]], rows.at[0], gsem0).wait()
        pltpu.make_async_copy(rows.at[1], acc_sum.at[didx.at[1]], ssem1).wait()
        pltpu.make_async_copy(onesv, acc_cnt.at[didx.at[1]], ssem1).wait()
        pltpu.make_async_copy(src2d.at[base], sidx.at[1], isem).wait()
        pltpu.make_async_copy(dst2d.at[base], didx.at[1], idsem).wait()

        plsc.subcore_barrier()
        pltpu.sync_copy(acc_sum.at[pl.ds(r0, RPT)], osum.at[pl.ds(r0, RPT)])
        pltpu.sync_copy(acc_cnt.at[pl.ds(r0, RPT)], ocnt.at[pl.ds(r0, RPT)])

    @pl.when(cid == 0)
    def _():
        run(srcC, dstC, tab_p, osum_c, ocnt_c)

    @pl.when(cid == 1)
    def _():
        run(srcP, dstP, tab_c, osum_p, ocnt_p)


@functools.lru_cache(maxsize=None)
def _seg_kernel():
    return pl.kernel(
        _seg_body,
        out_type=(jax.ShapeDtypeStruct((NPAD, 64), jnp.float32),
                  jax.ShapeDtypeStruct((NPAD, CW), jnp.float32),
                  jax.ShapeDtypeStruct((NPAD, 64), jnp.float32),
                  jax.ShapeDtypeStruct((NPAD, CW), jnp.float32)),
        mesh=_sc_mesh(),
        compiler_params=pltpu.CompilerParams(use_tc_tiling_on_sc=False),
        scratch_types=[
            pltpu.VMEM_SHARED((NPAD, 64), jnp.float32),
            pltpu.VMEM_SHARED((NPAD, CW), jnp.float32),
            pltpu.VMEM((2, ECHUNK), jnp.int32),
            pltpu.VMEM((2, ECHUNK), jnp.int32),
            pltpu.VMEM((2, ECHUNK, 64), jnp.float32),
            pltpu.VMEM((ECHUNK, CW), jnp.float32),
            pltpu.SemaphoreType.DMA,
            pltpu.SemaphoreType.DMA,
            pltpu.SemaphoreType.DMA,
            pltpu.SemaphoreType.DMA,
            pltpu.SemaphoreType.DMA,
            pltpu.SemaphoreType.DMA,
        ],
    )


def _seg_call(tab_c, tab_p, srcC2, dstC2, srcP2, dstP2, zsum, zcnt, onesr):
    return _seg_kernel()(
        tab_c, tab_p, srcC2, dstC2, srcP2, dstP2, zsum, zcnt, onesr)


def _dot(a, b):
    return jnp.dot(a, b, preferred_element_type=jnp.float32)


def _sig(x):
    return 0.5 * jnp.tanh(0.5 * x) + 0.5


def _accum_stats(b, acc, st, hs):
    @pl.when(b == 0)
    def _():
        acc[...] = jnp.zeros((4, 64), jnp.float32)

    hc, hp = hs
    acc[0:1, :] += jnp.sum(hc, axis=0, keepdims=True)
    acc[1:2, :] += jnp.sum(hc * hc, axis=0, keepdims=True)
    acc[2:3, :] += jnp.sum(hp, axis=0, keepdims=True)
    acc[3:4, :] += jnp.sum(hp * hp, axis=0, keepdims=True)
    st[...] = acc[...]


def _a1_body(xc, ec, xp, ep, wca, wce, bca, wpa, wpe, bpa,
             h0c, c0c, h0p, c0p, st, acc):
    def node_type(x, e, wa, we, b, ho, co):
        pre = _dot(x[...], wa[...]) + _dot(e[...], we[...]) + b[...]
        c0 = _sig(pre[:, 0:64]) * jnp.tanh(pre[:, 64:128])
        h0 = _sig(pre[:, 128:192]) * jnp.tanh(c0)
        co[...] = c0
        ho[...] = h0
        return h0

    h_c = node_type(xc, ec, wca, wce, bca, h0c, c0c)
    h_p = node_type(xp, ep, wpa, wpe, bpa, h0p, c0p)
    _accum_stats(pl.program_id(0), acc, st, (h_c, h_p))


def _graph_norm(h, prm, st, k):
    ms = prm[3 * k:3 * k + 1, :]
    w = prm[3 * k + 1:3 * k + 2, :]
    b = prm[3 * k + 2:3 * k + 3, :]
    inv_n = jnp.float32(1.0 / N)
    m = st[2 * k:2 * k + 1, :] * inv_n
    msq = st[2 * k + 1:2 * k + 2, :] * inv_n
    var = msq - m * m * (2.0 * ms - ms * ms)
    out = h - ms * m
    return out / jnp.sqrt(var + 1e-5) * w + b


def _b1_body(h0c, c0c, sc, cc, h0p, c0p, sp, cp_, wxc, wmc, bc, wxp, wmp, bp,
             nrm0, st0, c1c, h1c, c1p, h1p, st, acc):
    def affine(k):
        ms = nrm0[3 * k:3 * k + 1, :]
        w = nrm0[3 * k + 1:3 * k + 2, :]
        bb = nrm0[3 * k + 2:3 * k + 3, :]
        inv_n = jnp.float32(1.0 / N)
        m = st0[2 * k:2 * k + 1, :] * inv_n
        msq = st0[2 * k + 1:2 * k + 2, :] * inv_n
        var = msq - m * m * (2.0 * ms - ms * ms)
        istd = 1.0 / jnp.sqrt(var + 1e-5)
        alpha = istd * w
        return alpha, bb - ms * m * alpha

    def node_type(h0, c0, s, cnt, wx, wm, b, c1o, h1o, k):
        alpha, beta = affine(k)
        alpha_s, beta_s = affine(1 - k)
        x1 = h0[...] * alpha + beta
        cntv = cnt[...][:, 0:1]
        inv = 1.0 / jnp.maximum(cntv, 1.0)
        nonempty = cntv * inv
        mean = s[...] * inv * alpha_s + nonempty * beta_s
        pre = _dot(x1, wx[...]) + _dot(mean, wm[...]) + b[...]
        c1 = (_sig(pre[:, 64:128]) * c0[...]
              + _sig(pre[:, 0:64]) * jnp.tanh(pre[:, 128:192]))
        h1 = _sig(pre[:, 192:256]) * jnp.tanh(c1)
        c1o[...] = c1
        h1o[...] = h1
        return h1

    h_c = node_type(h0c, c0c, sc, cc, wxc, wmc, bc, c1c, h1c, 0)
    h_p = node_type(h0p, c0p, sp, cp_, wxp, wmp, bp, c1p, h1p, 1)
    _accum_stats(pl.program_id(0), acc, st, (h_c, h_p))


def _b2_body(h1c, h1p, nrm, st, wvt, bv, wot, bo, w1, b1, w2, b2,
             pred, hc, hp):
    xc = _graph_norm(h1c[...], nrm[...], st[...], 0)
    hp[...] = _graph_norm(h1p[...], nrm[...], st[...], 1)
    hc[...] = xc
    v = _dot(xc, wvt[...]) + bv[...]
    att = xc + _dot(v, wot[...]) + bo[...]
    z = jnp.maximum(_dot(att, w1[...]) + b1[...], 0.0)
    pred[...] = _dot(z, w2[...]) + b2[...]


def _full(shape):
    return pl.BlockSpec(shape, lambda b: (0, 0))


def _rows(width):
    return pl.BlockSpec((RB, width), lambda b: (b, 0))


def kernel(x_customer, x_product, edge_index_buys, edge_index_rev_buys, params):
    p = params
    f32 = jnp.float32

    country_ids = x_customer[:, -1].astype(jnp.int32)
    desc_ids = x_product[:, -1].astype(jnp.int32)
    x14 = x_customer[:, :-1]
    x4 = x_product[:, :-1]
    ctab = jnp.pad(p['country_embed'], ((0, 0), (0, 12)))
    tab = jnp.concatenate([ctab, p['desc_embed']])
    idx2d = jnp.concatenate([
        jnp.pad(country_ids, (0, IPAD - N)),
        jnp.pad(desc_ids + ctab.shape[0], (0, IPAD - N)),
    ]).reshape(NW * 16, 112)

    emb = _emb_call(idx2d, tab)
    emb_c = emb[:N]
    emb_p = emb[IPAD:IPAD + N]

    G0 = ('i', 'c', 'o')
    L0 = p['layers'][0]
    wca = jnp.concatenate([L0['W_' + g]['customer'][:14] for g in G0], axis=1)
    wce = jnp.pad(
        jnp.concatenate([L0['W_' + g]['customer'][14:] for g in G0], axis=1),
        ((0, 12), (0, 0)))
    bca = jnp.concatenate(
        [L0['b_' + g]['customer'] + L0['conv_' + g]['rev']['bl'][None, :]
         for g in G0], axis=1)
    wpa = jnp.concatenate([L0['W_' + g]['product'][:4] for g in G0], axis=1)
    wpe = jnp.concatenate([L0['W_' + g]['product'][4:] for g in G0], axis=1)
    bpa = jnp.concatenate(
        [L0['b_' + g]['product'] + L0['conv_' + g]['buys']['bl'][None, :]
         for g in G0], axis=1)

    h0c, c0c, h0p, c0p, st0 = pl.pallas_call(
        _a1_body,
        grid=(N // RB,),
        in_specs=[
            _rows(14), _rows(16), _rows(4), _rows(16),
            _full((14, 192)), _full((16, 192)), _full((1, 192)),
            _full((4, 192)), _full((16, 192)), _full((1, 192)),
        ],
        out_specs=[_rows(64)] * 4 + [_full((4, 64))],
        out_shape=[jax.ShapeDtypeStruct((N, 64), f32)] * 4
        + [jax.ShapeDtypeStruct((4, 64), f32)],
        scratch_shapes=[pltpu.VMEM((4, 64), f32)],
    )(x14, emb_c, x4, emb_p, wca, wce, bca, wpa, wpe, bpa)

    def norm_pack(layer):
        return jnp.stack([
            layer['norm']['customer']['mean_scale'],
            layer['norm']['customer']['weight'],
            layer['norm']['customer']['bias'],
            layer['norm']['product']['mean_scale'],
            layer['norm']['product']['weight'],
            layer['norm']['product']['bias'],
        ])

    def prep_edges(src, dst):
        s = jnp.concatenate([src, jnp.zeros((EPAD - E,), jnp.int32)])
        d = jnp.concatenate([dst, jnp.full((EPAD - E,), N, jnp.int32)])
        return s.reshape(NCHUNK, ECHUNK), d.reshape(NCHUNK, ECHUNK)

    srcC2, dstC2 = prep_edges(edge_index_rev_buys[0], edge_index_rev_buys[1])
    srcP2, dstP2 = prep_edges(edge_index_buys[0], edge_index_buys[1])
    zsum = jnp.zeros((RPT, 64), f32)
    zcnt = jnp.zeros((RPT, CW), f32)
    onesr = jnp.ones((ECHUNK, CW), f32)

    sum_c, cnt_c, sum_p, cnt_p = _seg_call(
        h0c, h0p, srcC2, dstC2, srcP2, dstP2, zsum, zcnt, onesr)

    sum_c, cnt_c = sum_c[:N], cnt_c[:N]
    sum_p, cnt_p = sum_p[:N], cnt_p[:N]

    L1 = p['layers'][1]
    G1 = ('i', 'f', 'c', 'o')

    def wpack(nt, et):
        wx = jnp.concatenate(
            [L1['W_' + g][nt] + L1['conv_' + g][et]['Wr'] for g in G1], axis=1)
        wm = jnp.concatenate([L1['conv_' + g][et]['Wl'] for g in G1], axis=1)
        b = jnp.concatenate(
            [L1['b_' + g][nt] + L1['conv_' + g][et]['bl'][None, :]
             for g in G1], axis=1)
        return wx, wm, b

    wxc, wmc, bc1 = wpack('customer', 'rev')
    wxp, wmp, bp1 = wpack('product', 'buys')

    c1c, h1c, c1p, h1p, st1 = pl.pallas_call(
        _b1_body,
        grid=(N // RB,),
        in_specs=[
            _rows(64), _rows(64), _rows(64), _rows(CW),
            _rows(64), _rows(64), _rows(64), _rows(CW),
            _full((64, 256)), _full((64, 256)), _full((1, 256)),
            _full((64, 256)), _full((64, 256)), _full((1, 256)),
            _full((6, 64)), _full((4, 64)),
        ],
        out_specs=[_rows(64)] * 4 + [_full((4, 64))],
        out_shape=[jax.ShapeDtypeStruct((N, 64), f32)] * 4
        + [jax.ShapeDtypeStruct((4, 64), f32)],
        scratch_shapes=[pltpu.VMEM((4, 64), f32)],
    )(h0c, c0c, sum_c, cnt_c, h0p, c0p, sum_p, cnt_p,
      wxc, wmc, bc1, wxp, wmp, bp1, norm_pack(L0), st0)

    wvt = p['mha']['in_w'][128:192].T
    bv = p['mha']['in_b'][128:192][None, :]
    wot = p['mha']['out_w'].T
    bo = p['mha']['out_b'][None, :]
    w1 = p['pred']['W1']
    b1 = p['pred']['b1'][None, :]
    w2 = jnp.pad(p['pred']['W2'], ((0, 0), (0, 7)))
    b2 = jnp.pad(p['pred']['b2'], (0, 7))[None, :]

    pred8, hc, hp = pl.pallas_call(
        _b2_body,
        grid=(N // RB,),
        in_specs=[
            _rows(64), _rows(64), _full((6, 64)), _full((4, 64)),
            _full((64, 64)), _full((1, 64)), _full((64, 64)), _full((1, 64)),
            _full((64, 32)), _full((1, 32)), _full((32, 8)), _full((1, 8)),
        ],
        out_specs=[_rows(8), _rows(64), _rows(64)],
        out_shape=[jax.ShapeDtypeStruct((N, 8), f32),
                   jax.ShapeDtypeStruct((N, 64), f32),
                   jax.ShapeDtypeStruct((N, 64), f32)],
    )(h1c, h1p, norm_pack(L1), st1, wvt, bv, wot, bo, w1, b1, w2, b2)

    return (pred8[:, :1], hc, hp, c1c, c1p)
